# baseline (speedup 1.0000x reference)
import functools

import jax
import jax.numpy as jnp
from jax import lax
from jax.experimental import pallas as pl
from jax.experimental.pallas import tpu as pltpu
from jax.experimental.pallas import tpu_sc as plsc

_LANES = 128
_ROWS_PER_BLOCK = 2048


def _sc_gather_build(BN, C):
    info = plsc.get_sparse_core_info()
    NC, NS, L = info.num_cores, info.num_subcores, info.num_lanes
    NW = NC * NS
    assert BN % (8 * NW) == 0
    chunk = BN // NW
    mesh = plsc.VectorSubcoreMesh(core_axis_name="c", subcore_axis_name="s")

    @functools.partial(
        pl.kernel, mesh=mesh,
        compiler_params=pltpu.CompilerParams(use_tc_tiling_on_sc=False),
        out_type=jax.ShapeDtypeStruct((BN, 1), jnp.float32),
        scratch_types=[
            pltpu.VMEM((chunk,), jnp.int32),
            pltpu.VMEM((chunk,), jnp.int32),
            pltpu.VMEM((chunk, 1), jnp.float32),
            pltpu.SemaphoreType.DMA,
        ],
    )
    def sc_gather(tg_hbm, x_hbm, z_hbm, tg_v, idx_v, rows_v, sem):
        wid = lax.axis_index("s") * NC + lax.axis_index("c")
        base = wid * chunk
        pltpu.sync_copy(tg_hbm.at[pl.ds(base, chunk)], tg_v)

        def body(j, carry):
            t = tg_v[pl.ds(j * L, L)]
            r = base + j * L + lax.iota(jnp.int32, L)
            idx_v[pl.ds(j * L, L)] = r * C + jnp.maximum(t - 1, 0)
            return carry

        lax.fori_loop(0, chunk // L, body, 0)
        pltpu.async_copy(x_hbm.at[idx_v], rows_v, sem).wait()
        pltpu.sync_copy(rows_v, z_hbm.at[pl.ds(base, chunk)])

    return sc_gather


def _fcos_body(xa_ref, z_ref, tg_ref, rpt_ref, rtt_ref, cn_ref, out_ref):
    i = pl.program_id(0)

    xb = xa_ref[...]
    e = jnp.exp(-jnp.abs(xb))
    u = 1.0 / (1.0 + e)
    sig = jnp.where(xb >= 0, u, 1.0 - u)
    sp = jnp.maximum(xb, 0.0) + jnp.log1p(e)
    f0sum = jnp.sum(sp * sig * sig)

    tg = tg_ref[...]
    posf = (tg > 0).astype(jnp.float32)
    npos = jnp.sum(posf)
    z = z_ref[...]
    ez = jnp.exp(-jnp.abs(z))
    lgz = jnp.log1p(ez)
    uz = 1.0 / (1.0 + ez)
    sigz = jnp.where(z >= 0, uz, 1.0 - uz)
    signz = 1.0 - sigz
    spz = jnp.maximum(z, 0.0) + lgz
    spnz = jnp.maximum(-z, 0.0) + lgz
    corr = 0.25 * spnz * signz * signz - 0.75 * spz * sigz * sigz
    fsum = 0.75 * f0sum + jnp.sum(corr * posf)

    p0 = rpt_ref[0]; p1 = rpt_ref[1]; p2 = rpt_ref[2]; p3 = rpt_ref[3]
    t0 = rtt_ref[0]; t1 = rtt_ref[1]; t2 = rtt_ref[2]; t3 = rtt_ref[3]
    lr_min = jnp.minimum(t0, t2); lr_max = jnp.maximum(t0, t2)
    tb_min = jnp.minimum(t1, t3); tb_max = jnp.maximum(t1, t3)
    cness_t = jnp.sqrt(lr_min / lr_max * (tb_min / tb_max))

    x1 = -p0; y1 = -p1; x2 = p2; y2 = p3
    x1g = -t0; y1g = -t1; x2g = t2; y2g = t3
    xi1 = jnp.maximum(x1, x1g); yi1 = jnp.maximum(y1, y1g)
    xi2 = jnp.minimum(x2, x2g); yi2 = jnp.minimum(y2, y2g)
    inter = jnp.where((yi2 > yi1) & (xi2 > xi1), (xi2 - xi1) * (yi2 - yi1), 0.0)
    union = (x2 - x1) * (y2 - y1) + (x2g - x1g) * (y2g - y1g) - inter
    iou = inter / (union + 1e-7)
    xc1 = jnp.minimum(x1, x1g); yc1 = jnp.minimum(y1, y1g)
    xc2 = jnp.maximum(x2, x2g); yc2 = jnp.maximum(y2, y2g)
    diag = (xc2 - xc1) ** 2 + (yc2 - yc1) ** 2 + 1e-7
    cdist = ((x1 + x2) / 2.0 - (x1g + x2g) / 2.0) ** 2 + \
            ((y1 + y2) / 2.0 - (y1g + y2g) / 2.0) ** 2
    diou = 1.0 - iou + cdist / diag
    w = cness_t * posf
    rnum = jnp.sum(diou * w)
    rden = jnp.sum(w)

    cn = cn_ref[...]
    bce = jnp.maximum(cn, 0.0) - cn * cness_t + jnp.log1p(jnp.exp(-jnp.abs(cn)))
    csum = jnp.sum(bce * posf)

    @pl.when(i == 0)
    def _init():
        out_ref[0] = fsum
        out_ref[1] = npos
        out_ref[2] = rnum
        out_ref[3] = rden
        out_ref[4] = csum

    @pl.when(i > 0)
    def _acc():
        out_ref[0] += fsum
        out_ref[1] += npos
        out_ref[2] += rnum
        out_ref[3] += rden
        out_ref[4] += csum


def kernel(cls_logits, reg_preds, cness_preds, cls_tgts, reg_tgts):
    B, N, C = cls_logits.shape
    BN = B * N
    R = _ROWS_PER_BLOCK
    assert BN % R == 0 and (BN * C) % (R * C) == 0 and (R * C) % _LANES == 0
    grid = BN // R
    S = R // _LANES
    RA = (R * C) // _LANES

    tg_flat = cls_tgts.reshape(BN).astype(jnp.int32)
    x_flat = cls_logits.reshape(BN * C, 1)
    z = _sc_gather_build(BN, C)(tg_flat, x_flat)

    xa = cls_logits.reshape((BN * C) // _LANES, _LANES)
    z2 = z.reshape(BN // _LANES, _LANES)
    tg2 = cls_tgts.reshape(BN // _LANES, _LANES).astype(jnp.int32)
    rpt = reg_preds.reshape(BN, 4).T.reshape(4, BN // _LANES, _LANES)
    rtt = reg_tgts.reshape(BN, 4).T.reshape(4, BN // _LANES, _LANES)
    cn = cness_preds.reshape(BN // _LANES, _LANES)

    partials = pl.pallas_call(
        _fcos_body,
        grid=(grid,),
        in_specs=[
            pl.BlockSpec((RA, _LANES), lambda i: (i, 0)),
            pl.BlockSpec((S, _LANES), lambda i: (i, 0)),
            pl.BlockSpec((S, _LANES), lambda i: (i, 0)),
            pl.BlockSpec((4, S, _LANES), lambda i: (0, i, 0)),
            pl.BlockSpec((4, S, _LANES), lambda i: (0, i, 0)),
            pl.BlockSpec((S, _LANES), lambda i: (i, 0)),
        ],
        out_specs=pl.BlockSpec(memory_space=pltpu.SMEM),
        out_shape=jax.ShapeDtypeStruct((8,), jnp.float32),
        compiler_params=pltpu.CompilerParams(
            dimension_semantics=("arbitrary",),
        ),
        interpret=False,
    )(xa, z2, tg2, rpt, rtt, cn)

    num_pos = partials[1]
    denom = jnp.maximum(num_pos, 1.0)
    cls_loss = partials[0] / denom
    reg_loss = partials[2] / (partials[3] + 1e-8)
    cness_loss = partials[4] / denom
    return cls_loss, reg_loss, cness_loss, cls_loss + reg_loss + cness_loss


# speedup vs baseline: 50.7050x; 50.7050x over previous
import jax
import jax.numpy as jnp
from jax.experimental import pallas as pl
from jax.experimental.pallas import tpu as pltpu

_LANES = 128
_ROWS_PER_BLOCK = 2048


def _fcos_body(x_ref, tg1_ref, tg2_ref, rpt_ref, rtt_ref, cn_ref, out_ref):
    i = pl.program_id(0)

    x = x_ref[...]
    tg1 = tg1_ref[...]
    cls_iota = jax.lax.broadcasted_iota(jnp.int32, x.shape, 1)
    m = (cls_iota == (tg1 - 1)) & (tg1 > 0)
    p = jax.nn.sigmoid(x)
    lg = jnp.log1p(jnp.exp(-jnp.abs(x)))
    ce = jnp.maximum(x, 0.0) - jnp.where(m, x, 0.0) + lg
    fac = jnp.where(m, 1.0 - p, p)
    alpha_t = jnp.where(m, 0.25, 0.75)
    fsum = jnp.sum(alpha_t * ce * fac * fac)

    tg2 = tg2_ref[...]
    posf = (tg2 > 0).astype(jnp.float32)
    npos = jnp.sum(posf)

    p0 = rpt_ref[0]; p1 = rpt_ref[1]; p2 = rpt_ref[2]; p3 = rpt_ref[3]
    t0 = rtt_ref[0]; t1 = rtt_ref[1]; t2 = rtt_ref[2]; t3 = rtt_ref[3]
    lr_min = jnp.minimum(t0, t2); lr_max = jnp.maximum(t0, t2)
    tb_min = jnp.minimum(t1, t3); tb_max = jnp.maximum(t1, t3)
    cness_t = jnp.sqrt(lr_min / lr_max * (tb_min / tb_max))

    x1 = -p0; y1 = -p1; x2 = p2; y2 = p3
    x1g = -t0; y1g = -t1; x2g = t2; y2g = t3
    xi1 = jnp.maximum(x1, x1g); yi1 = jnp.maximum(y1, y1g)
    xi2 = jnp.minimum(x2, x2g); yi2 = jnp.minimum(y2, y2g)
    inter = jnp.where((yi2 > yi1) & (xi2 > xi1), (xi2 - xi1) * (yi2 - yi1), 0.0)
    union = (x2 - x1) * (y2 - y1) + (x2g - x1g) * (y2g - y1g) - inter
    iou = inter / (union + 1e-7)
    xc1 = jnp.minimum(x1, x1g); yc1 = jnp.minimum(y1, y1g)
    xc2 = jnp.maximum(x2, x2g); yc2 = jnp.maximum(y2, y2g)
    diag = (xc2 - xc1) ** 2 + (yc2 - yc1) ** 2 + 1e-7
    cdist = ((x1 + x2) / 2.0 - (x1g + x2g) / 2.0) ** 2 + \
            ((y1 + y2) / 2.0 - (y1g + y2g) / 2.0) ** 2
    diou = 1.0 - iou + cdist / diag
    w = cness_t * posf
    rnum = jnp.sum(diou * w)
    rden = jnp.sum(w)

    cn = cn_ref[...]
    bce = jnp.maximum(cn, 0.0) - cn * cness_t + jnp.log1p(jnp.exp(-jnp.abs(cn)))
    csum = jnp.sum(bce * posf)

    @pl.when(i == 0)
    def _init():
        out_ref[0] = fsum
        out_ref[1] = npos
        out_ref[2] = rnum
        out_ref[3] = rden
        out_ref[4] = csum

    @pl.when(i > 0)
    def _acc():
        out_ref[0] += fsum
        out_ref[1] += npos
        out_ref[2] += rnum
        out_ref[3] += rden
        out_ref[4] += csum


def kernel(cls_logits, reg_preds, cness_preds, cls_tgts, reg_tgts):
    B, N, C = cls_logits.shape
    BN = B * N
    R = _ROWS_PER_BLOCK
    assert BN % R == 0 and BN % _LANES == 0
    grid = BN // R
    S = R // _LANES

    x = cls_logits.reshape(BN, C)
    tg1 = cls_tgts.reshape(BN, 1).astype(jnp.int32)
    tg2 = cls_tgts.reshape(BN // _LANES, _LANES).astype(jnp.int32)
    rpt = reg_preds.reshape(BN, 4).T.reshape(4, BN // _LANES, _LANES)
    rtt = reg_tgts.reshape(BN, 4).T.reshape(4, BN // _LANES, _LANES)
    cn = cness_preds.reshape(BN // _LANES, _LANES)

    partials = pl.pallas_call(
        _fcos_body,
        grid=(grid,),
        in_specs=[
            pl.BlockSpec((R, C), lambda i: (i, 0)),
            pl.BlockSpec((R, 1), lambda i: (i, 0)),
            pl.BlockSpec((S, _LANES), lambda i: (i, 0)),
            pl.BlockSpec((4, S, _LANES), lambda i: (0, i, 0)),
            pl.BlockSpec((4, S, _LANES), lambda i: (0, i, 0)),
            pl.BlockSpec((S, _LANES), lambda i: (i, 0)),
        ],
        out_specs=pl.BlockSpec(memory_space=pltpu.SMEM),
        out_shape=jax.ShapeDtypeStruct((8,), jnp.float32),
        compiler_params=pltpu.CompilerParams(
            dimension_semantics=("arbitrary",),
        ),
        interpret=False,
    )(x, tg1, tg2, rpt, rtt, cn)

    num_pos = partials[1]
    denom = jnp.maximum(num_pos, 1.0)
    cls_loss = partials[0] / denom
    reg_loss = partials[2] / (partials[3] + 1e-8)
    cness_loss = partials[4] / denom
    return cls_loss, reg_loss, cness_loss, cls_loss + reg_loss + cness_loss
